# baseline (speedup 1.0000x reference)
import functools

import jax
import jax.numpy as jnp
from jax import lax
from jax.experimental import pallas as pl
from jax.experimental.pallas import tpu as pltpu
from jax.experimental.pallas import tpu_sc as plsc


def _make_gather(B, D, NC, NS):
    NW = NC * NS
    b_per_w = B // NW
    C = 64
    n_ch = b_per_w // C
    L = 16
    D2 = 2 * D

    mesh = plsc.VectorSubcoreMesh(core_axis_name="c", subcore_axis_name="s")

    @functools.partial(
        pl.kernel,
        out_type=jax.ShapeDtypeStruct((B, D), jnp.float32),
        mesh=mesh,
        scratch_types=[
            pltpu.VMEM((b_per_w,), jnp.int32),
            pltpu.VMEM((n_ch, C), jnp.int32),
            pltpu.VMEM((b_per_w,), jnp.int32),
            pltpu.VMEM((C, D2), jnp.float32),
            pltpu.VMEM((C, D2), jnp.float32),
            pltpu.VMEM((C, D), jnp.float32),
            pltpu.VMEM((C, D), jnp.float32),
            pltpu.SemaphoreType.DMA,
            pltpu.SemaphoreType.DMA,
        ],
        compiler_params=pltpu.CompilerParams(needs_layout_passes=False),
    )
    def gather_kernel(idx_hbm, t2_hbm, out_hbm, idx_v, pid_v, off_v,
                      pairs_a, pairs_b, rows_a, rows_b, g_sem, w_sem):
        wid = lax.axis_index("s") * NC + lax.axis_index("c")
        base = wid * b_per_w
        pltpu.sync_copy(idx_hbm.at[pl.ds(base, b_per_w)], idx_v)
        iota = jnp.arange(L, dtype=jnp.int32)
        for ch in range(n_ch):
            for g in range(C // L):
                v = idx_v[pl.ds(ch * C + g * L, L)]
                pid_v[ch, pl.ds(g * L, L)] = lax.shift_right_logical(v, 1)
                off_v[pl.ds(ch * C + g * L, L)] = (v & 1) * D

        pairs = (pairs_a, pairs_b)
        rows = (rows_a, rows_b)

        def select_rows(ch, pairs_v, rows_v):
            def body(k, _):
                o16 = plsc.load_gather(
                    off_v, [jnp.full((L,), ch * C, jnp.int32) + k]
                )
                k16 = jnp.full((L,), 0, jnp.int32) + k
                for cg in range(D // L):
                    val = plsc.load_gather(
                        pairs_v, [k16, o16 + (iota + cg * L)]
                    )
                    rows_v[k, pl.ds(cg * L, L)] = val
                return 0

            lax.fori_loop(0, C, body, 0)

        copies = [pltpu.async_copy(t2_hbm.at[pid_v.at[0]], pairs_a, g_sem)]
        writes = []
        for ch in range(n_ch):
            if ch + 1 < n_ch:
                copies.append(
                    pltpu.async_copy(
                        t2_hbm.at[pid_v.at[ch + 1]], pairs[(ch + 1) % 2], g_sem
                    )
                )
            copies[ch].wait()
            if ch >= 2:
                writes[ch - 2].wait()
            select_rows(ch, pairs[ch % 2], rows[ch % 2])
            writes.append(
                pltpu.async_copy(
                    rows[ch % 2], out_hbm.at[pl.ds(base + ch * C, C)], w_sem
                )
            )
        writes[-2].wait()
        writes[-1].wait()

    return gather_kernel


def kernel(word_inputs, hidden, table):
    B = word_inputs.shape[0]
    V, D = table.shape
    t2 = table.reshape(V // 2, 2 * D)
    info = plsc.get_sparse_core_info()
    gather = _make_gather(B, D, info.num_cores, info.num_subcores)
    out = gather(word_inputs, t2)
    return (out.reshape(B, 1, D), hidden)
